# Initial kernel scaffold; baseline (speedup 1.0000x reference)
import jax
import jax.numpy as jnp
from jax.experimental import pallas as pl


def kernel(pred, target):
    raise NotImplementedError("write your pallas kernel here")


# speedup vs baseline: 9.6834x; 9.6834x over previous
import functools

import jax
import jax.numpy as jnp
from jax import lax
from jax.experimental import pallas as pl
from jax.experimental.pallas import tpu as pltpu

_THRESH = 0.7
_MIN_KEPT = 100000

_LANES = 128
_BLK = 512

_ONE_F32_BITS = 0x3F800000


def _ohem_body(nblk, x_ref, t_ref, out_ref, p_scr, nll_scr, acc_ref):
    i = pl.program_id(0)

    @pl.when(i == 0)
    def _init():
        acc_ref[0] = 0.0
        acc_ref[1] = 0.0

    x = x_ref[...]
    t = t_ref[...]
    cls = lax.broadcasted_iota(jnp.int32, x.shape, 0)
    g = jnp.sum(jnp.where(cls == t[None], x, 0.0), axis=0)
    m = jnp.max(x, axis=0)
    s = jnp.sum(jnp.exp(x - m[None]), axis=0)
    p = jnp.exp(g - m) / s
    nll = jnp.log(s) + (m - g)

    p_scr[pl.ds(i * _BLK, _BLK), :] = p
    nll_scr[pl.ds(i * _BLK, _BLK), :] = nll

    kept = p <= _THRESH
    acc_ref[0] += jnp.sum(kept.astype(jnp.float32))
    acc_ref[1] += jnp.sum(jnp.where(kept, nll, 0.0))

    @pl.when(i == nblk - 1)
    def _finish():
        cnt07 = acc_ref[0]

        @pl.when(cnt07 >= _MIN_KEPT)
        def _common():
            out_ref[...] = jnp.full((1, 1), acc_ref[1] / cnt07, jnp.float32)

        @pl.when(cnt07 < _MIN_KEPT)
        def _rare():
            pall = p_scr[...]

            def srch(_, c):
                lo, hi = c
                mid = (lo + hi) // 2
                thr = lax.bitcast_convert_type(mid, jnp.float32)
                cnt = jnp.sum((pall <= thr).astype(jnp.int32))
                ge = cnt >= _MIN_KEPT
                return (jnp.where(ge, lo, mid + 1), jnp.where(ge, mid, hi))

            _, hi = lax.fori_loop(
                0, 31, srch, (jnp.int32(0), jnp.int32(_ONE_F32_BITS))
            )
            thr = lax.bitcast_convert_type(hi, jnp.float32)
            keptk = pall <= thr
            kcnt = jnp.sum(keptk.astype(jnp.float32))
            ksum = jnp.sum(jnp.where(keptk, nll_scr[...], 0.0))
            out_ref[...] = jnp.full(
                (1, 1), ksum / jnp.maximum(kcnt, 1.0), jnp.float32
            )


@jax.jit
def kernel(pred, target):
    n, c = pred.shape
    rows = n // _LANES
    nblk = rows // _BLK
    x_t = pred.T.reshape(c, rows, _LANES)
    t2 = target.astype(jnp.int32).reshape(rows, _LANES)

    out = pl.pallas_call(
        functools.partial(_ohem_body, nblk),
        grid=(nblk,),
        in_specs=[
            pl.BlockSpec((c, _BLK, _LANES), lambda i: (0, i, 0)),
            pl.BlockSpec((_BLK, _LANES), lambda i: (i, 0)),
        ],
        out_specs=pl.BlockSpec((1, 1), lambda i: (0, 0)),
        out_shape=jax.ShapeDtypeStruct((1, 1), jnp.float32),
        scratch_shapes=[
            pltpu.VMEM((rows, _LANES), jnp.float32),
            pltpu.VMEM((rows, _LANES), jnp.float32),
            pltpu.SMEM((2,), jnp.float32),
        ],
    )(x_t, t2)
    return out[0, 0]
